# baseline (speedup 1.0000x reference)
import jax
import jax.numpy as jnp
from jax.experimental import pallas as pl

_BN = 1024


def _vq_kernel(x_ref, ct_ref, out_ref):
    x = x_ref[...]
    ct = ct_ref[...]
    scores = jnp.dot(x, ct, preferred_element_type=jnp.float32)
    c_sq = jnp.sum(ct * ct, axis=0)
    d2 = c_sq[None, :] - 2.0 * scores
    k = d2.shape[1]
    m = jnp.min(d2, axis=1, keepdims=True)
    iota = jax.lax.broadcasted_iota(jnp.int32, d2.shape, 1)
    idx = jnp.min(jnp.where(d2 == m, iota, k), axis=1)
    out_ref[...] = idx


def kernel(x, centroids):
    n, d = x.shape
    k = centroids.shape[0]
    ct = centroids.T
    return pl.pallas_call(
        _vq_kernel,
        grid=(n // _BN,),
        in_specs=[
            pl.BlockSpec((_BN, d), lambda i: (i, 0)),
            pl.BlockSpec((d, k), lambda i: (0, 0)),
        ],
        out_specs=pl.BlockSpec((_BN,), lambda i: (i,)),
        out_shape=jax.ShapeDtypeStruct((n,), jnp.int32),
    )(x, ct)


# speedup vs baseline: 1.5977x; 1.5977x over previous
import jax
import jax.numpy as jnp
from jax.experimental import pallas as pl

_BN = 1024


_KC = 128


def _vq_kernel(x_ref, ct2_ref, out_ref):
    x = x_ref[...]
    k = ct2_ref.shape[1]
    nchunks = k // _KC
    bn = x.shape[0]
    lane = jax.lax.broadcasted_iota(
        jnp.int32, (bn, _KC), 1).astype(jnp.float32)
    run_val = jnp.full((bn, _KC), jnp.inf, jnp.float32)
    run_idx = jnp.zeros((bn, _KC), jnp.float32)
    for j in range(nchunks):
        ct2j = ct2_ref[:, j * _KC:(j + 1) * _KC]
        csqj = 0.25 * jnp.sum(ct2j * ct2j, axis=0, keepdims=True)
        s = jnp.dot(x, ct2j, preferred_element_type=jnp.float32)
        d = s + csqj
        mask = d < run_val
        run_val = jnp.minimum(run_val, d)
        run_idx = jnp.where(mask, lane + float(j * _KC), run_idx)
    m = jnp.min(run_val, axis=1, keepdims=True)
    cand = jnp.where(run_val == m, run_idx, float(k))
    idx = jnp.min(cand, axis=1, keepdims=True)
    out_ref[...] = idx.astype(jnp.int32)


def kernel(x, centroids):
    n, d = x.shape
    k = centroids.shape[0]
    ct2 = -2.0 * centroids.T
    return pl.pallas_call(
        _vq_kernel,
        grid=(n // _BN,),
        in_specs=[
            pl.BlockSpec((_BN, d), lambda i: (i, 0)),
            pl.BlockSpec((d, k), lambda i: (0, 0)),
        ],
        out_specs=pl.BlockSpec((_BN, 1), lambda i: (i, 0)),
        out_shape=jax.ShapeDtypeStruct((n, 1), jnp.int32),
    )(x, ct2).reshape(n)


# speedup vs baseline: 2.1079x; 1.3194x over previous
import jax
import jax.numpy as jnp
from jax.experimental import pallas as pl

_BN = 4096


_KC = 128


def _vq_kernel(x_ref, ct2_ref, out_ref):
    x = x_ref[...]
    k = ct2_ref.shape[1]
    nchunks = k // _KC
    bn = x.shape[0]
    lane = jax.lax.broadcasted_iota(
        jnp.int32, (bn, _KC), 1).astype(jnp.float32)
    run_val = jnp.full((bn, _KC), jnp.inf, jnp.float32)
    run_idx = jnp.zeros((bn, _KC), jnp.float32)
    for j in range(nchunks):
        ct2j = ct2_ref[:, j * _KC:(j + 1) * _KC]
        csqj = 0.25 * jnp.sum(ct2j * ct2j, axis=0, keepdims=True)
        s = jnp.dot(x, ct2j, preferred_element_type=jnp.float32)
        d = s + csqj
        mask = d < run_val
        run_val = jnp.minimum(run_val, d)
        run_idx = jnp.where(mask, lane + float(j * _KC), run_idx)
    m = jnp.min(run_val, axis=1, keepdims=True)
    cand = jnp.where(run_val == m, run_idx, float(k))
    idx = jnp.min(cand, axis=1, keepdims=True)
    out_ref[...] = idx.astype(jnp.int32)


def kernel(x, centroids):
    n, d = x.shape
    k = centroids.shape[0]
    ct2 = -2.0 * centroids.T
    return pl.pallas_call(
        _vq_kernel,
        grid=(n // _BN,),
        in_specs=[
            pl.BlockSpec((_BN, d), lambda i: (i, 0)),
            pl.BlockSpec((d, k), lambda i: (0, 0)),
        ],
        out_specs=pl.BlockSpec((_BN, 1), lambda i: (i, 0)),
        out_shape=jax.ShapeDtypeStruct((n, 1), jnp.int32),
    )(x, ct2).reshape(n)


# speedup vs baseline: 3.4845x; 1.6530x over previous
import jax
import jax.numpy as jnp
from jax.experimental import pallas as pl

_BN = 4096
_KC = 128


def _vq_kernel(xt_ref, ct2_ref, out_ref):
    xt = xt_ref[...]
    k = ct2_ref.shape[1]
    nchunks = k // _KC
    bn = xt.shape[1]
    lane = jax.lax.broadcasted_iota(
        jnp.int32, (bn, _KC), 1).astype(jnp.float32)
    run_val = jnp.full((bn, _KC), jnp.inf, jnp.float32)
    run_idx = jnp.zeros((bn, _KC), jnp.float32)
    for j in range(nchunks):
        ct2j = ct2_ref[:, j * _KC:(j + 1) * _KC]
        csqj = 0.25 * jnp.sum(ct2j * ct2j, axis=0, keepdims=True)
        s = jax.lax.dot_general(
            xt, ct2j, (((0,), (0,)), ((), ())),
            preferred_element_type=jnp.float32)
        d = s + csqj
        mask = d < run_val
        run_val = jnp.minimum(run_val, d)
        run_idx = jnp.where(mask, lane + float(j * _KC), run_idx)
    m = jnp.min(run_val, axis=1, keepdims=True)
    cand = jnp.where(run_val == m, run_idx, float(k))
    idx = jnp.min(cand, axis=1)
    out_ref[...] = idx.astype(jnp.int32).reshape(1, bn // 128, 128)


def kernel(x, centroids):
    n, d = x.shape
    k = centroids.shape[0]
    xt = x.T
    ct2 = -2.0 * centroids.T
    nb = n // _BN
    out = pl.pallas_call(
        _vq_kernel,
        grid=(nb,),
        in_specs=[
            pl.BlockSpec((d, _BN), lambda i: (0, i)),
            pl.BlockSpec((d, k), lambda i: (0, 0)),
        ],
        out_specs=pl.BlockSpec((1, _BN // 128, 128), lambda i: (i, 0, 0)),
        out_shape=jax.ShapeDtypeStruct((nb, _BN // 128, 128), jnp.int32),
    )(xt, ct2)
    return out.reshape(n)


# speedup vs baseline: 3.8586x; 1.1074x over previous
import jax
import jax.numpy as jnp
from jax.experimental import pallas as pl

_BN = 4096
_KC = 128


def _vq_kernel(xt_ref, ct2_ref, out_ref):
    xt = xt_ref[...]
    k = ct2_ref.shape[1]
    nchunks = k // _KC
    bn = xt.shape[1]
    lane = jax.lax.broadcasted_iota(
        jnp.int32, (bn, _KC), 1).astype(jnp.float32)
    ds = []
    for j in range(nchunks):
        ct2j = ct2_ref[:, j * _KC:(j + 1) * _KC]
        csqj = 0.25 * jnp.sum(ct2j * ct2j, axis=0, keepdims=True)
        s = jax.lax.dot_general(
            xt, ct2j, (((0,), (0,)), ((), ())),
            preferred_element_type=jnp.float32)
        ds.append(s + csqj)
    vals = [(d, lane + float(j * _KC)) for j, d in enumerate(ds)]
    while len(vals) > 1:
        nxt = []
        for (v0, i0), (v1, i1) in zip(vals[::2], vals[1::2]):
            c = v1 < v0
            nxt.append((jnp.minimum(v0, v1), jnp.where(c, i1, i0)))
        vals = nxt
    run_val, run_idx = vals[0]
    m = jnp.min(run_val, axis=1, keepdims=True)
    cand = jnp.where(run_val == m, run_idx, float(k))
    idx = jnp.min(cand, axis=1)
    out_ref[...] = idx.astype(jnp.int32).reshape(1, bn // 128, 128)


def kernel(x, centroids):
    n, d = x.shape
    k = centroids.shape[0]
    xt = x.T
    ct2 = -2.0 * centroids.T
    nb = n // _BN
    out = pl.pallas_call(
        _vq_kernel,
        grid=(nb,),
        in_specs=[
            pl.BlockSpec((d, _BN), lambda i: (0, i)),
            pl.BlockSpec((d, k), lambda i: (0, 0)),
        ],
        out_specs=pl.BlockSpec((1, _BN // 128, 128), lambda i: (i, 0, 0)),
        out_shape=jax.ShapeDtypeStruct((nb, _BN // 128, 128), jnp.int32),
    )(xt, ct2)
    return out.reshape(n)


# speedup vs baseline: 4.0238x; 1.0428x over previous
import jax
import jax.numpy as jnp
from jax.experimental import pallas as pl
from jax.experimental.pallas import tpu as pltpu

_BN = 4096
_KC = 128
_DA = 72


def _vq_kernel(xt_ref, ct2_ref, out_ref, xa_ref):
    d = xt_ref.shape[0]
    k = ct2_ref.shape[1]
    bn = xt_ref.shape[1]
    nchunks = k // _KC
    xa_ref[:d, :] = xt_ref[...]
    sub = jax.lax.broadcasted_iota(jnp.int32, (_DA - d, bn), 0)
    xa_ref[d:, :] = jnp.where(sub == 0, 1.0, 0.0)
    xa = xa_ref[...]
    lane = jax.lax.broadcasted_iota(
        jnp.int32, (bn, _KC), 1).astype(jnp.float32)
    ds = []
    for j in range(nchunks):
        ct2j = ct2_ref[:, j * _KC:(j + 1) * _KC]
        ds.append(jax.lax.dot_general(
            xa, ct2j, (((0,), (0,)), ((), ())),
            preferred_element_type=jnp.float32))
    vals = [(dj, lane + float(j * _KC)) for j, dj in enumerate(ds)]
    while len(vals) > 1:
        nxt = []
        for (v0, i0), (v1, i1) in zip(vals[::2], vals[1::2]):
            c = v1 < v0
            nxt.append((jnp.minimum(v0, v1), jnp.where(c, i1, i0)))
        vals = nxt
    run_val, run_idx = vals[0]
    m = jnp.min(run_val, axis=1, keepdims=True)
    cand = jnp.where(run_val == m, run_idx, float(k))
    idx = jnp.min(cand, axis=1)
    out_ref[...] = idx.reshape(1, bn // 128, 128).astype(jnp.int32)


def kernel(x, centroids):
    n, d = x.shape
    k = centroids.shape[0]
    xt = x.T
    csq = jnp.sum(centroids * centroids, axis=1)
    ct2a = jnp.concatenate(
        [-2.0 * centroids.T, csq[None, :],
         jnp.zeros((_DA - d - 1, k), jnp.float32)], axis=0)
    nb = n // _BN
    out = pl.pallas_call(
        _vq_kernel,
        grid=(nb,),
        in_specs=[
            pl.BlockSpec((d, _BN), lambda i: (0, i)),
            pl.BlockSpec((_DA, k), lambda i: (0, 0)),
        ],
        out_specs=pl.BlockSpec((1, _BN // 128, 128), lambda i: (i, 0, 0)),
        out_shape=jax.ShapeDtypeStruct((nb, _BN // 128, 128), jnp.int32),
        scratch_shapes=[pltpu.VMEM((_DA, _BN), jnp.float32)],
    )(xt, ct2a)
    return out.reshape(n)


# speedup vs baseline: 4.0961x; 1.0180x over previous
import jax
import jax.numpy as jnp
from jax.experimental import pallas as pl

_BN = 4096
_KC = 128
_FBIAS = 0x4B000000


def _vq_kernel(xt_ref, c2_ref, csq_ref, out_ref):
    xt = xt_ref[...]
    k = c2_ref.shape[0]
    bn = xt.shape[1]
    nchunks = k // _KC
    ds = []
    for j in range(nchunks):
        c2j = c2_ref[j * _KC:(j + 1) * _KC, :]
        csqj = csq_ref[j * _KC:(j + 1) * _KC, :]
        s = jax.lax.dot_general(
            c2j, xt, (((1,), (0,)), ((), ())),
            preferred_element_type=jnp.float32)
        ds.append(s + csqj)
    row = jax.lax.broadcasted_iota(jnp.int32, (_KC, bn), 0)
    vals = [(dj, row + (j * _KC + _FBIAS)) for j, dj in enumerate(ds)]
    while len(vals) > 1:
        nxt = []
        for (v0, i0), (v1, i1) in zip(vals[::2], vals[1::2]):
            c = v1 < v0
            nxt.append((jnp.minimum(v0, v1), jnp.where(c, i1, i0)))
        vals = nxt
    run_val, run_idx = vals[0]
    m = jnp.min(run_val, axis=0, keepdims=True)
    ibits = jax.lax.bitcast_convert_type(run_idx, jnp.float32)
    big = jax.lax.bitcast_convert_type(jnp.int32(_FBIAS + 512), jnp.float32)
    cand = jnp.where(run_val == m, ibits, big)
    idxf = jnp.min(cand, axis=0)
    idx = jax.lax.bitcast_convert_type(idxf, jnp.int32) - _FBIAS
    out_ref[...] = idx.reshape(1, bn // 128, 128)


def kernel(x, centroids):
    n, d = x.shape
    k = centroids.shape[0]
    xt = x.T
    c2 = -2.0 * centroids
    csq = jnp.sum(centroids * centroids, axis=1)[:, None]
    nb = n // _BN
    out = pl.pallas_call(
        _vq_kernel,
        grid=(nb,),
        in_specs=[
            pl.BlockSpec((d, _BN), lambda i: (0, i)),
            pl.BlockSpec((k, d), lambda i: (0, 0)),
            pl.BlockSpec((k, 1), lambda i: (0, 0)),
        ],
        out_specs=pl.BlockSpec((1, _BN // 128, 128), lambda i: (i, 0, 0)),
        out_shape=jax.ShapeDtypeStruct((nb, _BN // 128, 128), jnp.int32),
    )(xt, c2, csq)
    return out.reshape(n)


# speedup vs baseline: 4.7869x; 1.1687x over previous
import jax
import jax.numpy as jnp
from jax.experimental import pallas as pl

_BN = 4096
_KC = 128
_FBIAS = 0x4B000000


def _vq_kernel(xt_ref, c2_ref, csq_ref, out_ref):
    xt = xt_ref[...]
    k = c2_ref.shape[0]
    bn = xt.shape[1]
    nchunks = k // _KC
    s = jax.lax.dot_general(
        c2_ref[...], xt, (((1,), (0,)), ((), ())),
        preferred_element_type=jnp.float32)
    d = s + csq_ref[...]
    ds = [d[j * _KC:(j + 1) * _KC, :] for j in range(nchunks)]
    row = jax.lax.broadcasted_iota(jnp.int32, (_KC, bn), 0)
    vals = [(dj, row + (j * _KC + _FBIAS)) for j, dj in enumerate(ds)]
    while len(vals) > 1:
        nxt = []
        for (v0, i0), (v1, i1) in zip(vals[::2], vals[1::2]):
            c = v1 < v0
            nxt.append((jnp.minimum(v0, v1), jnp.where(c, i1, i0)))
        vals = nxt
    run_val, run_idx = vals[0]
    m = jnp.min(run_val, axis=0, keepdims=True)
    ibits = jax.lax.bitcast_convert_type(run_idx, jnp.float32)
    big = jax.lax.bitcast_convert_type(jnp.int32(_FBIAS + 512), jnp.float32)
    cand = jnp.where(run_val == m, ibits, big)
    idxf = jnp.min(cand, axis=0)
    idx = jax.lax.bitcast_convert_type(idxf, jnp.int32) - _FBIAS
    out_ref[...] = idx.reshape(1, bn // 128, 128)


def kernel(x, centroids):
    n, d = x.shape
    k = centroids.shape[0]
    xt = x.T
    c2 = -2.0 * centroids
    csq = jnp.sum(centroids * centroids, axis=1)[:, None]
    nb = n // _BN
    out = pl.pallas_call(
        _vq_kernel,
        grid=(nb,),
        in_specs=[
            pl.BlockSpec((d, _BN), lambda i: (0, i)),
            pl.BlockSpec((k, d), lambda i: (0, 0)),
            pl.BlockSpec((k, 1), lambda i: (0, 0)),
        ],
        out_specs=pl.BlockSpec((1, _BN // 128, 128), lambda i: (i, 0, 0)),
        out_shape=jax.ShapeDtypeStruct((nb, _BN // 128, 128), jnp.int32),
    )(xt, c2, csq)
    return out.reshape(n)


# speedup vs baseline: 5.0604x; 1.0571x over previous
import jax
import jax.numpy as jnp
from jax.experimental import pallas as pl

_BN = 8192
_KC = 128
_FBIAS = 0x4B000000


def _vq_kernel(xt_ref, c2_ref, csq_ref, out_ref):
    xt = xt_ref[...]
    k = c2_ref.shape[0]
    bn = xt.shape[1]
    nchunks = k // _KC
    s = jax.lax.dot_general(
        c2_ref[...], xt, (((1,), (0,)), ((), ())),
        preferred_element_type=jnp.float32)
    d = s + csq_ref[...]
    ds = [d[j * _KC:(j + 1) * _KC, :] for j in range(nchunks)]
    row = jax.lax.broadcasted_iota(jnp.int32, (_KC, bn), 0)
    vals = [(dj, row + (j * _KC + _FBIAS)) for j, dj in enumerate(ds)]
    while len(vals) > 1:
        nxt = []
        for (v0, i0), (v1, i1) in zip(vals[::2], vals[1::2]):
            c = v1 < v0
            nxt.append((jnp.minimum(v0, v1), jnp.where(c, i1, i0)))
        vals = nxt
    run_val, run_idx = vals[0]
    m = jnp.min(run_val, axis=0, keepdims=True)
    ibits = jax.lax.bitcast_convert_type(run_idx, jnp.float32)
    big = jax.lax.bitcast_convert_type(jnp.int32(_FBIAS + 512), jnp.float32)
    cand = jnp.where(run_val == m, ibits, big)
    idxf = jnp.min(cand, axis=0)
    idx = jax.lax.bitcast_convert_type(idxf, jnp.int32) - _FBIAS
    out_ref[...] = idx.reshape(1, bn // 128, 128)


def kernel(x, centroids):
    n, d = x.shape
    k = centroids.shape[0]
    xt = x.T
    c2 = -2.0 * centroids
    csq = jnp.sum(centroids * centroids, axis=1)[:, None]
    nb = n // _BN
    out = pl.pallas_call(
        _vq_kernel,
        grid=(nb,),
        in_specs=[
            pl.BlockSpec((d, _BN), lambda i: (0, i)),
            pl.BlockSpec((k, d), lambda i: (0, 0)),
            pl.BlockSpec((k, 1), lambda i: (0, 0)),
        ],
        out_specs=pl.BlockSpec((1, _BN // 128, 128), lambda i: (i, 0, 0)),
        out_shape=jax.ShapeDtypeStruct((nb, _BN // 128, 128), jnp.int32),
    )(xt, c2, csq)
    return out.reshape(n)


# speedup vs baseline: 5.1098x; 1.0098x over previous
import jax
import jax.numpy as jnp
from jax.experimental import pallas as pl

_BN = 16384
_KC = 128
_FBIAS = 0x4B000000


def _vq_kernel(xt_ref, c2_ref, csq_ref, out_ref):
    xt = xt_ref[...]
    k = c2_ref.shape[0]
    bn = xt.shape[1]
    nchunks = k // _KC
    s = jax.lax.dot_general(
        c2_ref[...], xt, (((1,), (0,)), ((), ())),
        preferred_element_type=jnp.float32)
    d = s + csq_ref[...]
    ds = [d[j * _KC:(j + 1) * _KC, :] for j in range(nchunks)]
    row = jax.lax.broadcasted_iota(jnp.int32, (_KC, bn), 0)
    vals = [(dj, row + (j * _KC + _FBIAS)) for j, dj in enumerate(ds)]
    while len(vals) > 1:
        nxt = []
        for (v0, i0), (v1, i1) in zip(vals[::2], vals[1::2]):
            c = v1 < v0
            nxt.append((jnp.minimum(v0, v1), jnp.where(c, i1, i0)))
        vals = nxt
    run_val, run_idx = vals[0]
    m = jnp.min(run_val, axis=0, keepdims=True)
    ibits = jax.lax.bitcast_convert_type(run_idx, jnp.float32)
    big = jax.lax.bitcast_convert_type(jnp.int32(_FBIAS + 512), jnp.float32)
    cand = jnp.where(run_val == m, ibits, big)
    idxf = jnp.min(cand, axis=0)
    idx = jax.lax.bitcast_convert_type(idxf, jnp.int32) - _FBIAS
    out_ref[...] = idx.reshape(1, bn // 128, 128)


def kernel(x, centroids):
    n, d = x.shape
    k = centroids.shape[0]
    xt = x.T
    c2 = -2.0 * centroids
    csq = jnp.sum(centroids * centroids, axis=1)[:, None]
    nb = n // _BN
    out = pl.pallas_call(
        _vq_kernel,
        grid=(nb,),
        in_specs=[
            pl.BlockSpec((d, _BN), lambda i: (0, i)),
            pl.BlockSpec((k, d), lambda i: (0, 0)),
            pl.BlockSpec((k, 1), lambda i: (0, 0)),
        ],
        out_specs=pl.BlockSpec((1, _BN // 128, 128), lambda i: (i, 0, 0)),
        out_shape=jax.ShapeDtypeStruct((nb, _BN // 128, 128), jnp.int32),
    )(xt, c2, csq)
    return out.reshape(n)


# speedup vs baseline: 5.1700x; 1.0118x over previous
import jax
import jax.numpy as jnp
from jax.experimental import pallas as pl

_BN = 16384
_KC = 128
_FBIAS = 0x4B000000


def _vq_kernel(xt_ref, ct_ref, csq_ref, out_ref):
    xt = xt_ref[...]
    k = ct_ref.shape[1]
    bn = xt.shape[1]
    nchunks = k // _KC
    ct2 = ct_ref[...] * -2.0
    s = jax.lax.dot_general(
        ct2, xt, (((0,), (0,)), ((), ())),
        preferred_element_type=jnp.float32)
    d = s + csq_ref[...]
    ds = [d[j * _KC:(j + 1) * _KC, :] for j in range(nchunks)]
    row = jax.lax.broadcasted_iota(jnp.int32, (_KC, bn), 0)
    vals = [(dj, row + (j * _KC + _FBIAS)) for j, dj in enumerate(ds)]
    while len(vals) > 1:
        nxt = []
        for (v0, i0), (v1, i1) in zip(vals[::2], vals[1::2]):
            c = v1 < v0
            nxt.append((jnp.minimum(v0, v1), jnp.where(c, i1, i0)))
        vals = nxt
    run_val, run_idx = vals[0]
    m = jnp.min(run_val, axis=0, keepdims=True)
    ibits = jax.lax.bitcast_convert_type(run_idx, jnp.float32)
    big = jax.lax.bitcast_convert_type(jnp.int32(_FBIAS + 512), jnp.float32)
    cand = jnp.where(run_val == m, ibits, big)
    idxf = jnp.min(cand, axis=0)
    idx = jax.lax.bitcast_convert_type(idxf, jnp.int32) - _FBIAS
    out_ref[...] = idx.reshape(1, bn // 128, 128)


def kernel(x, centroids):
    n, d = x.shape
    k = centroids.shape[0]
    xt = x.T
    ct = centroids.T
    csq = jnp.sum(centroids * centroids, axis=1)[:, None]
    nb = n // _BN
    out = pl.pallas_call(
        _vq_kernel,
        grid=(nb,),
        in_specs=[
            pl.BlockSpec((d, _BN), lambda i: (0, i)),
            pl.BlockSpec((d, k), lambda i: (0, 0)),
            pl.BlockSpec((k, 1), lambda i: (0, 0)),
        ],
        out_specs=pl.BlockSpec((1, _BN // 128, 128), lambda i: (i, 0, 0)),
        out_shape=jax.ShapeDtypeStruct((nb, _BN // 128, 128), jnp.int32),
    )(xt, ct, csq)
    return out.reshape(n)


# speedup vs baseline: 5.3292x; 1.0308x over previous
import jax
import jax.numpy as jnp
from jax.experimental import pallas as pl

_BN = 16384
_KC = 128
_FBIAS = 0x4B000000


def _vq_kernel(xt_ref, ct_ref, out_ref):
    xt = xt_ref[...]
    dd = ct_ref.shape[0]
    k = ct_ref.shape[1]
    bn = xt.shape[1]
    nchunks = k // _KC
    ctv = ct_ref[...]
    ct2 = ctv * -2.0
    csq = jax.lax.dot_general(
        ctv * ctv, jnp.ones((dd, 1), jnp.float32),
        (((0,), (0,)), ((), ())),
        preferred_element_type=jnp.float32)
    s = jax.lax.dot_general(
        ct2, xt, (((0,), (0,)), ((), ())),
        preferred_element_type=jnp.float32)
    d = s + csq
    ds = [d[j * _KC:(j + 1) * _KC, :] for j in range(nchunks)]
    row = jax.lax.broadcasted_iota(jnp.int32, (_KC, bn), 0)
    vals = [(dj, row + (j * _KC + _FBIAS)) for j, dj in enumerate(ds)]
    while len(vals) > 1:
        nxt = []
        for (v0, i0), (v1, i1) in zip(vals[::2], vals[1::2]):
            c = v1 < v0
            nxt.append((jnp.minimum(v0, v1), jnp.where(c, i1, i0)))
        vals = nxt
    run_val, run_idx = vals[0]
    m = jnp.min(run_val, axis=0, keepdims=True)
    ibits = jax.lax.bitcast_convert_type(run_idx, jnp.float32)
    big = jax.lax.bitcast_convert_type(jnp.int32(_FBIAS + 512), jnp.float32)
    cand = jnp.where(run_val == m, ibits, big)
    idxf = jnp.min(cand, axis=0)
    idx = jax.lax.bitcast_convert_type(idxf, jnp.int32) - _FBIAS
    out_ref[...] = idx.reshape(1, bn // 128, 128)


def kernel(x, centroids):
    n, d = x.shape
    k = centroids.shape[0]
    xt = x.T
    ct = centroids.T
    nb = n // _BN
    out = pl.pallas_call(
        _vq_kernel,
        grid=(nb,),
        in_specs=[
            pl.BlockSpec((d, _BN), lambda i: (0, i)),
            pl.BlockSpec((d, k), lambda i: (0, 0)),
        ],
        out_specs=pl.BlockSpec((1, _BN // 128, 128), lambda i: (i, 0, 0)),
        out_shape=jax.ShapeDtypeStruct((nb, _BN // 128, 128), jnp.int32),
    )(xt, ct)
    return out.reshape(n)


# speedup vs baseline: 5.3543x; 1.0047x over previous
import jax
import jax.numpy as jnp
from jax.experimental import pallas as pl

_BN = 16384
_KC = 128
_FBIAS = 0x4B000000


def _vq_kernel(xt_ref, ct_ref, out_ref):
    xt = xt_ref[...]
    dd = ct_ref.shape[0]
    k = ct_ref.shape[1]
    bn = xt.shape[1]
    nchunks = k // _KC
    ctv = ct_ref[...]
    ct2 = ctv * -2.0
    csq = jnp.transpose(
        jnp.sum(ctv * ctv, axis=0, keepdims=True))
    s = jax.lax.dot_general(
        ct2, xt, (((0,), (0,)), ((), ())),
        preferred_element_type=jnp.float32)
    d = s + csq
    ds = [d[j * _KC:(j + 1) * _KC, :] for j in range(nchunks)]
    row = jax.lax.broadcasted_iota(jnp.int32, (_KC, bn), 0)
    vals = [(dj, row + (j * _KC + _FBIAS)) for j, dj in enumerate(ds)]
    while len(vals) > 1:
        nxt = []
        for (v0, i0), (v1, i1) in zip(vals[::2], vals[1::2]):
            c = v1 < v0
            nxt.append((jnp.minimum(v0, v1), jnp.where(c, i1, i0)))
        vals = nxt
    run_val, run_idx = vals[0]
    m = jnp.min(run_val, axis=0, keepdims=True)
    ibits = jax.lax.bitcast_convert_type(run_idx, jnp.float32)
    big = jax.lax.bitcast_convert_type(jnp.int32(_FBIAS + 512), jnp.float32)
    cand = jnp.where(run_val == m, ibits, big)
    idxf = jnp.min(cand, axis=0)
    idx = jax.lax.bitcast_convert_type(idxf, jnp.int32) - _FBIAS
    out_ref[...] = idx.reshape(1, bn // 128, 128)


def kernel(x, centroids):
    n, d = x.shape
    k = centroids.shape[0]
    xt = x.T
    ct = centroids.T
    nb = n // _BN
    out = pl.pallas_call(
        _vq_kernel,
        grid=(nb,),
        in_specs=[
            pl.BlockSpec((d, _BN), lambda i: (0, i)),
            pl.BlockSpec((d, k), lambda i: (0, 0)),
        ],
        out_specs=pl.BlockSpec((1, _BN // 128, 128), lambda i: (i, 0, 0)),
        out_shape=jax.ShapeDtypeStruct((nb, _BN // 128, 128), jnp.int32),
    )(xt, ct)
    return out.reshape(n)


# speedup vs baseline: 5.8829x; 1.0987x over previous
import jax
import jax.numpy as jnp
from jax.experimental import pallas as pl

_BN = 16384
_KC = 64
_FBIAS = 0x4B000000


def _vq_kernel(xt_ref, ct_ref, out_ref):
    xt = xt_ref[...]
    dd = ct_ref.shape[0]
    k = ct_ref.shape[1]
    bn = xt.shape[1]
    nchunks = k // _KC
    ctv = ct_ref[...]
    ct2 = ctv * -2.0
    csq = jnp.transpose(
        jnp.sum(ctv * ctv, axis=0, keepdims=True))
    s = jax.lax.dot_general(
        ct2, xt, (((0,), (0,)), ((), ())),
        preferred_element_type=jnp.float32)
    d = s + csq
    ds = [d[j * _KC:(j + 1) * _KC, :] for j in range(nchunks)]
    row = jax.lax.broadcasted_iota(jnp.int32, (_KC, bn), 0)
    vals = [(dj, row + (j * _KC + _FBIAS)) for j, dj in enumerate(ds)]
    while len(vals) > 1:
        nxt = []
        for (v0, i0), (v1, i1) in zip(vals[::2], vals[1::2]):
            c = v1 < v0
            nxt.append((jnp.minimum(v0, v1), jnp.where(c, i1, i0)))
        vals = nxt
    run_val, run_idx = vals[0]
    m = jnp.min(run_val, axis=0, keepdims=True)
    ibits = jax.lax.bitcast_convert_type(run_idx, jnp.float32)
    big = jax.lax.bitcast_convert_type(jnp.int32(_FBIAS + 512), jnp.float32)
    cand = jnp.where(run_val == m, ibits, big)
    idxf = jnp.min(cand, axis=0)
    idx = jax.lax.bitcast_convert_type(idxf, jnp.int32) - _FBIAS
    out_ref[...] = idx.reshape(1, bn // 128, 128)


def kernel(x, centroids):
    n, d = x.shape
    k = centroids.shape[0]
    xt = x.T
    ct = centroids.T
    nb = n // _BN
    out = pl.pallas_call(
        _vq_kernel,
        grid=(nb,),
        in_specs=[
            pl.BlockSpec((d, _BN), lambda i: (0, i)),
            pl.BlockSpec((d, k), lambda i: (0, 0)),
        ],
        out_specs=pl.BlockSpec((1, _BN // 128, 128), lambda i: (i, 0, 0)),
        out_shape=jax.ShapeDtypeStruct((nb, _BN // 128, 128), jnp.int32),
    )(xt, ct)
    return out.reshape(n)


# speedup vs baseline: 5.9552x; 1.0123x over previous
import jax
import jax.numpy as jnp
from jax.experimental import pallas as pl

_BN = 16384
_KC = 32
_FBIAS = 0x4B000000


def _vq_kernel(xt_ref, ct_ref, out_ref):
    xt = xt_ref[...]
    dd = ct_ref.shape[0]
    k = ct_ref.shape[1]
    bn = xt.shape[1]
    nchunks = k // _KC
    ctv = ct_ref[...]
    ct2 = ctv * -2.0
    csq = jnp.transpose(
        jnp.sum(ctv * ctv, axis=0, keepdims=True))
    s = jax.lax.dot_general(
        ct2, xt, (((0,), (0,)), ((), ())),
        preferred_element_type=jnp.float32)
    d = s + csq
    ds = [d[j * _KC:(j + 1) * _KC, :] for j in range(nchunks)]
    row = jax.lax.broadcasted_iota(jnp.int32, (_KC, bn), 0)
    vals = [(dj, row + (j * _KC + _FBIAS)) for j, dj in enumerate(ds)]
    while len(vals) > 1:
        nxt = []
        for (v0, i0), (v1, i1) in zip(vals[::2], vals[1::2]):
            c = v1 < v0
            nxt.append((jnp.minimum(v0, v1), jnp.where(c, i1, i0)))
        vals = nxt
    run_val, run_idx = vals[0]
    m = jnp.min(run_val, axis=0, keepdims=True)
    ibits = jax.lax.bitcast_convert_type(run_idx, jnp.float32)
    big = jax.lax.bitcast_convert_type(jnp.int32(_FBIAS + 512), jnp.float32)
    cand = jnp.where(run_val == m, ibits, big)
    idxf = jnp.min(cand, axis=0)
    idx = jax.lax.bitcast_convert_type(idxf, jnp.int32) - _FBIAS
    out_ref[...] = idx.reshape(1, bn // 128, 128)


def kernel(x, centroids):
    n, d = x.shape
    k = centroids.shape[0]
    xt = x.T
    ct = centroids.T
    nb = n // _BN
    out = pl.pallas_call(
        _vq_kernel,
        grid=(nb,),
        in_specs=[
            pl.BlockSpec((d, _BN), lambda i: (0, i)),
            pl.BlockSpec((d, k), lambda i: (0, 0)),
        ],
        out_specs=pl.BlockSpec((1, _BN // 128, 128), lambda i: (i, 0, 0)),
        out_shape=jax.ShapeDtypeStruct((nb, _BN // 128, 128), jnp.int32),
    )(xt, ct)
    return out.reshape(n)
